# baseline (speedup 1.0000x reference)
import functools

import jax
import jax.numpy as jnp
from jax import lax
from jax.experimental import pallas as pl
from jax.experimental.pallas import tpu as pltpu

_VOCAB = 1000000
_EMB = 64
_CTX = 200
_BLK = 16384
_NB = (_VOCAB + _BLK - 1) // _BLK
_SCALE = 1.0 / (2.0 * 100.0)


def _mv_body(idx_ref, emb_hbm, w_ref, lg_ref, bm_ref, bs_ref, gbuf, vbuf, gsem):
    i = pl.program_id(0)

    @pl.when(i == 0)
    def _prologue():
        for t in range(_CTX):
            pltpu.make_async_copy(
                emb_hbm.at[pl.ds(idx_ref[t], 1)], gbuf.at[pl.ds(t, 1)], gsem
            ).start()
        for t in range(_CTX):
            pltpu.make_async_copy(
                emb_hbm.at[pl.ds(0, 1)], gbuf.at[pl.ds(0, 1)], gsem
            ).wait()
        vbuf[...] = jnp.sum(gbuf[...], axis=0, keepdims=True) * _SCALE

    blk = lax.dot_general(
        vbuf[...],
        w_ref[...],
        (((1,), (1,)), ((), ())),
        preferred_element_type=jnp.float32,
    )
    lg_ref[...] = blk
    limit = _VOCAB - i * _BLK
    cols = lax.broadcasted_iota(jnp.int32, (1, _BLK), 1)
    mblk = jnp.where(cols < limit, blk, -1e30)
    bm = jnp.max(mblk, axis=1, keepdims=True)
    bs = jnp.sum(jnp.exp(mblk - bm), axis=1, keepdims=True)
    bm_ref[...] = jnp.broadcast_to(bm, (1, 128))
    bs_ref[...] = jnp.broadcast_to(bs, (1, 128))


def _norm_body(lg_ref, bm_ref, bs_ref, out_ref):
    gm = jnp.max(bm_ref[...], axis=1, keepdims=True)
    t = bs_ref[...] * jnp.exp(bm_ref[...] - gm)
    zz = jnp.sum(t, axis=1, keepdims=True) * (1.0 / 128.0)
    out_ref[...] = lg_ref[...] - (gm + jnp.log(zz))


def _tc_logits(idx, emb, w):
    logits, bm, bs = pl.pallas_call(
        _mv_body,
        grid=(_NB,),
        in_specs=[
            pl.BlockSpec(memory_space=pltpu.SMEM),
            pl.BlockSpec(memory_space=pl.ANY),
            pl.BlockSpec((_BLK, _EMB), lambda i: (i, 0)),
        ],
        out_specs=[
            pl.BlockSpec((1, _BLK), lambda i: (0, i)),
            pl.BlockSpec((1, 128), lambda i: (0, i)),
            pl.BlockSpec((1, 128), lambda i: (0, i)),
        ],
        out_shape=[
            jax.ShapeDtypeStruct((1, _NB * _BLK), jnp.float32),
            jax.ShapeDtypeStruct((1, _NB * 128), jnp.float32),
            jax.ShapeDtypeStruct((1, _NB * 128), jnp.float32),
        ],
        scratch_shapes=[
            pltpu.VMEM((_CTX, _EMB), jnp.float32),
            pltpu.VMEM((1, _EMB), jnp.float32),
            pltpu.SemaphoreType.DMA,
        ],
    )(idx, emb, w)
    out = pl.pallas_call(
        _norm_body,
        grid=(_NB,),
        in_specs=[
            pl.BlockSpec((1, _BLK), lambda i: (0, i)),
            pl.BlockSpec((1, _NB * 128), lambda i: (0, 0)),
            pl.BlockSpec((1, _NB * 128), lambda i: (0, 0)),
        ],
        out_specs=pl.BlockSpec((1, _BLK), lambda i: (0, i)),
        out_shape=jax.ShapeDtypeStruct((1, _NB * _BLK), jnp.float32),
    )(logits, bm, bs)
    return out[:, :_VOCAB]


def kernel(input, embedding_weight, out_weight):
    idx = input.astype(jnp.int32)
    return _tc_logits(idx, embedding_weight, out_weight)
